# Initial kernel scaffold; baseline (speedup 1.0000x reference)
import jax
import jax.numpy as jnp
from jax.experimental import pallas as pl


def kernel(group_idx, weight):
    raise NotImplementedError("write your pallas kernel here")


# speedup vs baseline: 3.9505x; 3.9505x over previous
import functools

import jax
import jax.numpy as jnp
from jax import lax
from jax.experimental import pallas as pl
from jax.experimental.pallas import tpu as pltpu
from jax.experimental.pallas import tpu_sc as plsc

DIM = 64
LANE = 128
CR = 4
CHUNK = CR * LANE


def kernel(group_idx, weight):
    b, s = group_idx.shape
    n = b * s
    idx2d = group_idx.reshape(n // LANE, LANE).astype(jnp.int32)

    info = plsc.get_sparse_core_info()
    nc, ns = info.num_cores, info.num_subcores
    nw = nc * ns
    r_per_w = (n // LANE) // nw
    n_chunks = r_per_w // CR

    mesh = plsc.VectorSubcoreMesh(core_axis_name="c", subcore_axis_name="s")

    @functools.partial(
        pl.kernel,
        mesh=mesh,
        out_type=jax.ShapeDtypeStruct((n, DIM), jnp.float32),
        scratch_types=[
            pltpu.VMEM((CR, LANE), jnp.int32),
            pltpu.VMEM((CHUNK, DIM), jnp.float32),
            pltpu.SemaphoreType.DMA,
        ],
        compiler_params=pltpu.CompilerParams(use_tc_tiling_on_sc=False),
    )
    def gather_kernel(table_hbm, idx_hbm, out_hbm, idx_v, rows_v, sem):
        wid = lax.axis_index("s") * nc + lax.axis_index("c")
        row_base = wid * r_per_w

        def body(g, carry):
            r0 = row_base + g * CR
            pltpu.sync_copy(idx_hbm.at[pl.ds(r0, CR)], idx_v)
            copies = [
                pltpu.async_copy(
                    table_hbm.at[idx_v.at[j]],
                    rows_v.at[pl.ds(j * LANE, LANE)],
                    sem,
                )
                for j in range(CR)
            ]
            for cp in copies:
                cp.wait()
            pltpu.sync_copy(rows_v, out_hbm.at[pl.ds(r0 * LANE, CHUNK)])
            return carry

        lax.fori_loop(0, n_chunks, body, 0)

    out = gather_kernel(weight, idx2d)
    return out.reshape(b, s, DIM)


# speedup vs baseline: 4.2482x; 1.0754x over previous
import functools

import jax
import jax.numpy as jnp
from jax import lax
from jax.experimental import pallas as pl
from jax.experimental.pallas import tpu as pltpu
from jax.experimental.pallas import tpu_sc as plsc

DIM = 64
LANE = 128
CR = 5
CHUNK = CR * LANE
NBUF = 2


def kernel(group_idx, weight):
    b, s = group_idx.shape
    n = b * s
    idx2d = group_idx.reshape(n // LANE, LANE).astype(jnp.int32)

    info = plsc.get_sparse_core_info()
    nc, ns = info.num_cores, info.num_subcores
    nw = nc * ns
    r_per_w = (n // LANE) // nw
    n_chunks = r_per_w // CR
    n_pairs = n_chunks // NBUF

    mesh = plsc.VectorSubcoreMesh(core_axis_name="c", subcore_axis_name="s")

    @functools.partial(
        pl.kernel,
        mesh=mesh,
        out_type=jax.ShapeDtypeStruct((n, DIM), jnp.float32),
        scratch_types=[
            pltpu.VMEM((r_per_w, LANE), jnp.int32),
            pltpu.VMEM((NBUF, CHUNK, DIM), jnp.float32),
            pltpu.SemaphoreType.DMA,
            pltpu.SemaphoreType.DMA,
            pltpu.SemaphoreType.DMA,
            pltpu.SemaphoreType.DMA,
        ],
        compiler_params=pltpu.CompilerParams(use_tc_tiling_on_sc=False),
    )
    def gather_kernel(table_hbm, idx_hbm, out_hbm,
                      idx_l, rows_v, sg0, sg1, so0, so1):
        wid = lax.axis_index("s") * nc + lax.axis_index("c")
        row_base = wid * r_per_w
        out_base = row_base * LANE
        sg = (sg0, sg1)
        so = (so0, so1)

        pltpu.sync_copy(idx_hbm.at[pl.ds(row_base, r_per_w)], idx_l)

        def issue_gathers(g, bb):
            for j in range(CR):
                pltpu.async_copy(
                    table_hbm.at[idx_l.at[g * CR + j]],
                    rows_v.at[bb].at[pl.ds(j * LANE, LANE)],
                    sg[bb],
                )

        def drain_gathers(bb):
            pltpu.make_async_copy(
                table_hbm.at[pl.ds(0, CHUNK)], rows_v.at[bb], sg[bb]
            ).wait()

        def drain_out(bb):
            pltpu.make_async_copy(
                rows_v.at[bb], out_hbm.at[pl.ds(0, CHUNK)], so[bb]
            ).wait()

        def slot(g, bb, b2):
            @pl.when(g + 1 < n_chunks)
            def _():
                @pl.when(g >= 1)
                def _():
                    drain_out(b2)
                issue_gathers(g + 1, b2)
            drain_gathers(bb)
            pltpu.async_copy(
                rows_v.at[bb],
                out_hbm.at[pl.ds(out_base + g * CHUNK, CHUNK)],
                so[bb],
            )

        issue_gathers(0, 0)

        def body(p, carry):
            g = p * NBUF
            slot(g, 0, 1)
            slot(g + 1, 1, 0)
            return carry

        lax.fori_loop(0, n_pairs, body, 0)
        drain_out(0)
        drain_out(1)

    out = gather_kernel(weight, idx2d)
    return out.reshape(b, s, DIM)


# speedup vs baseline: 4.2639x; 1.0037x over previous
import functools

import jax
import jax.numpy as jnp
from jax import lax
from jax.experimental import pallas as pl
from jax.experimental.pallas import tpu as pltpu
from jax.experimental.pallas import tpu_sc as plsc

DIM = 64
LANE = 128
CRI = 2
NBUF = 2


def kernel(group_idx, weight):
    b, s = group_idx.shape
    n = b * s
    idx_flat = group_idx.reshape(n).astype(jnp.int32)

    info = plsc.get_sparse_core_info()
    nc, ns = info.num_cores, info.num_subcores
    nw = nc * ns
    rows_w = b // nw
    n_chunks = rows_w // CRI
    n_pairs = n_chunks // NBUF
    chunk_n = CRI * s

    segs = []
    off = 0
    while off < s:
        ln = min(LANE, s - off)
        segs.append((off, ln))
        off += ln

    mesh = plsc.VectorSubcoreMesh(core_axis_name="c", subcore_axis_name="s")

    @functools.partial(
        pl.kernel,
        mesh=mesh,
        out_type=jax.ShapeDtypeStruct((b, s, DIM), jnp.float32),
        scratch_types=[
            pltpu.VMEM((rows_w * s,), jnp.int32),
            pltpu.VMEM((NBUF, CRI, s, DIM), jnp.float32),
            pltpu.SemaphoreType.DMA,
            pltpu.SemaphoreType.DMA,
            pltpu.SemaphoreType.DMA,
            pltpu.SemaphoreType.DMA,
        ],
        compiler_params=pltpu.CompilerParams(use_tc_tiling_on_sc=False),
    )
    def gather_kernel(table_hbm, idx_hbm, out_hbm,
                      idx_l, rows_v, sg0, sg1, so0, so1):
        wid = lax.axis_index("s") * nc + lax.axis_index("c")
        i_base = wid * rows_w
        sg = (sg0, sg1)
        so = (so0, so1)

        pltpu.sync_copy(idx_hbm.at[pl.ds(i_base * s, rows_w * s)], idx_l)

        def issue_gathers(g, bb):
            for r in range(CRI):
                for (o, ln) in segs:
                    pltpu.async_copy(
                        table_hbm.at[idx_l.at[pl.ds((g * CRI + r) * s + o, ln)]],
                        rows_v.at[bb, r, pl.ds(o, ln)],
                        sg[bb],
                    )

        def drain_gathers(bb):
            pltpu.make_async_copy(
                out_hbm.at[pl.ds(0, CRI)], rows_v.at[bb], sg[bb]
            ).wait()

        def drain_out(bb):
            pltpu.make_async_copy(
                rows_v.at[bb], out_hbm.at[pl.ds(0, CRI)], so[bb]
            ).wait()

        def slot(g, bb, b2):
            @pl.when(g + 1 < n_chunks)
            def _():
                @pl.when(g >= 1)
                def _():
                    drain_out(b2)
                issue_gathers(g + 1, b2)
            drain_gathers(bb)
            pltpu.async_copy(
                rows_v.at[bb],
                out_hbm.at[pl.ds(i_base + g * CRI, CRI)],
                so[bb],
            )

        issue_gathers(0, 0)

        def body(p, carry):
            g = p * NBUF
            slot(g, 0, 1)
            slot(g + 1, 1, 0)
            return carry

        lax.fori_loop(0, n_pairs, body, 0)
        drain_out(0)
        drain_out(1)

    return gather_kernel(weight, idx_flat)


# speedup vs baseline: 4.6776x; 1.0970x over previous
import functools

import jax
import jax.numpy as jnp
from jax import lax
from jax.experimental import pallas as pl
from jax.experimental.pallas import tpu as pltpu
from jax.experimental.pallas import tpu_sc as plsc

DIM = 64
PAD = 2 * DIM
LANE = 128
NBUF = 2
VL = 16


def kernel(group_idx, weight):
    b, s = group_idx.shape
    idx = group_idx.astype(jnp.int32)
    table = jnp.pad(weight, ((0, 0), (0, PAD - DIM)))

    info = plsc.get_sparse_core_info()
    nc, ns = info.num_cores, info.num_subcores
    nw = nc * ns
    rows_w = b // nw
    n_pairs = rows_w // NBUF

    segs = []
    off = 0
    while off < s:
        ln = min(LANE, s - off)
        segs.append((off, ln))
        off += ln

    mesh = plsc.VectorSubcoreMesh(core_axis_name="c", subcore_axis_name="s")

    @functools.partial(
        pl.kernel,
        mesh=mesh,
        out_type=jax.ShapeDtypeStruct((b, s, DIM), jnp.float32),
        scratch_types=[
            pltpu.VMEM((rows_w, s), jnp.int32),
            pltpu.VMEM((NBUF, s, PAD), jnp.float32),
            pltpu.VMEM((s, DIM), jnp.float32),
            pltpu.SemaphoreType.DMA,
            pltpu.SemaphoreType.DMA,
            pltpu.SemaphoreType.DMA,
        ],
        compiler_params=pltpu.CompilerParams(use_tc_tiling_on_sc=True),
    )
    def gather_kernel(table_hbm, idx_hbm, out_hbm,
                      idx_l, rows_v, c_v, sg0, sg1, so):
        wid = lax.axis_index("s") * nc + lax.axis_index("c")
        i_base = wid * rows_w
        sg = (sg0, sg1)

        pltpu.sync_copy(idx_hbm.at[pl.ds(i_base, rows_w)], idx_l)

        def issue_gathers(g, bb):
            for (o, ln) in segs:
                pltpu.async_copy(
                    table_hbm.at[idx_l.at[g, pl.ds(o, ln)]],
                    rows_v.at[bb, pl.ds(o, ln)],
                    sg[bb],
                )

        def drain_gathers(bb):
            pltpu.make_async_copy(
                table_hbm.at[pl.ds(0, s)], rows_v.at[bb], sg[bb]
            ).wait()

        def drain_out():
            pltpu.make_async_copy(c_v, out_hbm.at[0], so).wait()

        def compact(bb):
            def row(r, carry):
                for k in range(DIM // VL):
                    c_v[r, pl.ds(k * VL, VL)] = rows_v[bb, r, pl.ds(k * VL, VL)]
                return carry
            lax.fori_loop(0, s, row, 0)

        def slot(g, bb, b2):
            @pl.when(g + 1 < rows_w)
            def _():
                issue_gathers(g + 1, b2)
            drain_gathers(bb)
            @pl.when(g >= 1)
            def _():
                drain_out()
            compact(bb)
            pltpu.async_copy(c_v, out_hbm.at[i_base + g], so)

        issue_gathers(0, 0)

        def body(p, carry):
            g = p * NBUF
            slot(g, 0, 1)
            slot(g + 1, 1, 0)
            return carry

        lax.fori_loop(0, n_pairs, body, 0)
        drain_out()

    return gather_kernel(table, idx)


# speedup vs baseline: 4.9415x; 1.0564x over previous
import functools

import jax
import jax.numpy as jnp
from jax import lax
from jax.experimental import pallas as pl
from jax.experimental.pallas import tpu as pltpu
from jax.experimental.pallas import tpu_sc as plsc

DIM = 64
VL = 16
NBUF = 2
UNROLL = 8


def kernel(group_idx, weight):
    b, s = group_idx.shape
    v = weight.shape[0]
    idx_t = jnp.transpose(group_idx.astype(jnp.int32))
    w_t = jnp.transpose(weight)

    info = plsc.get_sparse_core_info()
    nc, ns = info.num_cores, info.num_subcores
    nw = nc * ns
    n_pass = DIM // nw
    n_jpairs = s // NBUF

    mesh = plsc.VectorSubcoreMesh(core_axis_name="c", subcore_axis_name="s")

    @functools.partial(
        pl.kernel,
        mesh=mesh,
        out_type=jax.ShapeDtypeStruct((s, DIM, b), jnp.float32),
        scratch_types=[
            pltpu.VMEM((v,), jnp.float32),
            pltpu.VMEM((NBUF, b), jnp.int32),
            pltpu.VMEM((NBUF, b), jnp.float32),
            pltpu.SemaphoreType.DMA,
            pltpu.SemaphoreType.DMA,
            pltpu.SemaphoreType.DMA,
            pltpu.SemaphoreType.DMA,
        ],
        compiler_params=pltpu.CompilerParams(
            use_tc_tiling_on_sc=True, needs_layout_passes=False
        ),
    )
    def gather_kernel(w_hbm, idx_hbm, out_hbm,
                      col_v, idx_v, res_v, si0, si1, sr0, sr1):
        wid = lax.axis_index("s") * nc + lax.axis_index("c")
        si = (si0, si1)
        sr = (sr0, sr1)

        def issue_idx(j, bb):
            pltpu.async_copy(idx_hbm.at[j], idx_v.at[bb], si[bb])

        def drain_idx(bb):
            pltpu.make_async_copy(idx_hbm.at[0], idx_v.at[bb], si[bb]).wait()

        def drain_res(bb):
            pltpu.make_async_copy(res_v.at[bb], out_hbm.at[0, 0], sr[bb]).wait()

        def gather_row(bb):
            def grp(g, carry):
                for u in range(UNROLL):
                    o = (g * UNROLL + u) * VL
                    ii = idx_v[bb, pl.ds(o, VL)]
                    res_v[bb, pl.ds(o, VL)] = plsc.load_gather(col_v, [ii])
                return carry
            lax.fori_loop(0, b // (VL * UNROLL), grp, 0)

        def do_pass(k):
            pltpu.sync_copy(w_hbm.at[k], col_v)
            issue_idx(0, 0)

            def slot(j, bb, b2):
                @pl.when(j + 1 < s)
                def _():
                    issue_idx(j + 1, b2)
                drain_idx(bb)
                @pl.when(j >= NBUF)
                def _():
                    drain_res(bb)
                gather_row(bb)
                pltpu.async_copy(res_v.at[bb], out_hbm.at[j, k], sr[bb])

            def body(p, carry):
                j = p * NBUF
                slot(j, 0, 1)
                slot(j + 1, 1, 0)
                return carry

            lax.fori_loop(0, n_jpairs, body, 0)
            drain_res(0)
            drain_res(1)

        for p in range(n_pass):
            do_pass(wid + p * nw)

    out_t = gather_kernel(w_t, idx_t)
    return jnp.transpose(out_t, (2, 0, 1))


# speedup vs baseline: 8.2414x; 1.6678x over previous
import functools

import jax
import jax.numpy as jnp
from jax import lax
from jax.experimental import pallas as pl
from jax.experimental.pallas import tpu as pltpu
from jax.experimental.pallas import tpu_sc as plsc

DIM = 64
VL = 16
NBUF = 2
UNROLL = 8


def kernel(group_idx, weight):
    b, s = group_idx.shape
    v = weight.shape[0]
    idx_t = jnp.transpose(group_idx.astype(jnp.int32))
    w_t = jnp.transpose(weight)

    info = plsc.get_sparse_core_info()
    nc, ns = info.num_cores, info.num_subcores
    nw = nc * ns
    n_pass = DIM // nw
    n_jpairs = s // NBUF

    mesh = plsc.VectorSubcoreMesh(core_axis_name="c", subcore_axis_name="s")

    @functools.partial(
        pl.kernel,
        mesh=mesh,
        out_type=jax.ShapeDtypeStruct((s, DIM, b), jnp.float32),
        scratch_types=[
            pltpu.VMEM((v,), jnp.float32),
            pltpu.VMEM((NBUF, b), jnp.int32),
            pltpu.VMEM((NBUF, b), jnp.float32),
            pltpu.SemaphoreType.DMA,
            pltpu.SemaphoreType.DMA,
            pltpu.SemaphoreType.DMA,
            pltpu.SemaphoreType.DMA,
        ],
        compiler_params=pltpu.CompilerParams(
            use_tc_tiling_on_sc=True, needs_layout_passes=False
        ),
    )
    def gather_kernel(w_hbm, idx_hbm, out_hbm,
                      col_v, idx_v, res_v, si0, si1, sr0, sr1):
        wid = lax.axis_index("s") * nc + lax.axis_index("c")
        si = (si0, si1)
        sr = (sr0, sr1)

        def issue_idx(j, bb):
            pltpu.async_copy(idx_hbm.at[j], idx_v.at[bb], si[bb])

        def drain_idx(bb):
            pltpu.make_async_copy(idx_hbm.at[0], idx_v.at[bb], si[bb]).wait()

        def drain_res(bb):
            pltpu.make_async_copy(res_v.at[bb], out_hbm.at[0, 0], sr[bb]).wait()

        def gather_row(bb):
            @plsc.parallel_loop(0, b, step=VL, unroll=UNROLL)
            def grp(o):
                ii = idx_v[bb, pl.ds(o, VL)]
                res_v[bb, pl.ds(o, VL)] = plsc.load_gather(col_v, [ii])

        def do_pass(k):
            pltpu.sync_copy(w_hbm.at[k], col_v)
            issue_idx(0, 0)

            def slot(j, bb, b2):
                @pl.when(j + 1 < s)
                def _():
                    issue_idx(j + 1, b2)
                drain_idx(bb)
                @pl.when(j >= NBUF)
                def _():
                    drain_res(bb)
                gather_row(bb)
                pltpu.async_copy(res_v.at[bb], out_hbm.at[j, k], sr[bb])

            def body(p, carry):
                j = p * NBUF
                slot(j, 0, 1)
                slot(j + 1, 1, 0)
                return carry

            lax.fori_loop(0, n_jpairs, body, 0)
            drain_res(0)
            drain_res(1)

        for p in range(n_pass):
            do_pass(wid + p * nw)

    out_t = gather_kernel(w_t, idx_t)
    return jnp.transpose(out_t, (2, 0, 1))
